# baseline (speedup 1.0000x reference)
import functools

import jax
import jax.numpy as jnp
from jax import lax
from jax.experimental import pallas as pl
from jax.experimental.pallas import tpu as pltpu
from jax.experimental.pallas import tpu_sc as plsc

_N = 10000
_NPAD = 10240
_E = 320000
_TRASH = _NPAD - 1
_NCORE = 2
_NSUB = 16
_NW = _NCORE * _NSUB
_EPT = _E // _NW
_EPT_PAD = 10240
_NB = _EPT_PAD // 128
_CH = 40
_NCH = _NB // _CH
_RPS = _NPAD // _NSUB

_NH = _NPAD // 2
_ACC_R = _NH + 128
_TRASH_L = _NH
_BPB = 4096
_TOTB = 2636
_TOT = _TOTB * 128
_ZPS = _NH // _NSUB


def _sc_agg(z, ridx2, cidx2, meta):
    mesh = plsc.VectorSubcoreMesh(core_axis_name="c", subcore_axis_name="s")

    @functools.partial(
        pl.kernel,
        out_type=jax.ShapeDtypeStruct((_NCORE, _NPAD, 128), jnp.float32),
        mesh=mesh,
        scratch_types=[
            pltpu.VMEM((128,), jnp.int32),
            pltpu.VMEM((8, 128), jnp.int32),
            pltpu.VMEM((8, 128), jnp.int32),
            pltpu.VMEM((128, 128), jnp.float32),
            pltpu.VMEM_SHARED((_NH, 128), jnp.float32),
            pltpu.VMEM_SHARED((_ACC_R, 128), jnp.float32),
        ],
    )
    def agg(z_hbm, ridx_hbm, cidx_hbm, meta_hbm,
            p_hbm, meta_v, ridx_v, cidx_v, buf, zsh, acc):
        c = lax.axis_index("c")
        s = lax.axis_index("s")
        wid = c * _NSUB + s

        pltpu.sync_copy(meta_hbm, meta_v)

        def zero_buf():
            def zrow(i, carry):
                for k in range(8):
                    buf[i, pl.ds(k * 16, 16)] = jnp.zeros((16,), jnp.float32)
                return carry
            lax.fori_loop(0, 128, zrow, 0)

        def zero_acc():
            for i in range(3):
                blk = i * _NSUB + s

                @pl.when(blk < _ACC_R // 128)
                def _():
                    pltpu.sync_copy(
                        buf, acc.at[pl.ds(pl.multiple_of(blk * 128, 128), 128)])

        zero_buf()
        zero_acc()
        plsc.subcore_barrier()

        zofs = pl.multiple_of(s * _ZPS, 8)
        for k in range(4):
            shalf, dhalf = k % 2, k // 2
            pltpu.sync_copy(
                z_hbm.at[pl.ds(pl.multiple_of(shalf * _NH + s * _ZPS, 8),
                               _ZPS)],
                zsh.at[pl.ds(zofs, _ZPS)])
            plsc.subcore_barrier()

            mv = meta_v[pl.ds(0, 16)]
            nch = mv[k]
            offc = mv[4 + k]
            nloc = (nch - wid + 31) // 32

            def chunk(i, carry):
                base = pl.multiple_of((offc + wid + i * 32) * 8, 8)
                pltpu.sync_copy(ridx_hbm.at[pl.ds(base, 8)], ridx_v)
                pltpu.sync_copy(cidx_hbm.at[pl.ds(base, 8)], cidx_v)
                for j in range(8):
                    pltpu.sync_copy(zsh.at[ridx_v.at[j]], buf)
                    pltpu.sync_copy(buf, acc.at[cidx_v.at[j]], add=True)
                return carry
            lax.fori_loop(0, nloc, chunk, 0)
            plsc.subcore_barrier()

            if k % 2 == 1:
                pltpu.sync_copy(
                    acc.at[pl.ds(zofs, _ZPS)],
                    p_hbm.at[c, pl.ds(pl.multiple_of(
                        dhalf * _NH + s * _ZPS, 8), _ZPS)])
                if k == 1:
                    plsc.subcore_barrier()
                    zero_buf()
                    zero_acc()

    return agg(z, ridx2, cidx2, meta)


def _sc_deg(cidx):
    mesh = plsc.VectorSubcoreMesh(core_axis_name="c", subcore_axis_name="s")

    @functools.partial(
        pl.kernel,
        out_type=jax.ShapeDtypeStruct((_NCORE, _NPAD, 128), jnp.float32),
        mesh=mesh,
        scratch_types=[
            pltpu.VMEM((_NB, 128), jnp.int32),
            pltpu.VMEM((128, 128), jnp.float32),
            pltpu.VMEM_SHARED((_NPAD, 128), jnp.float32),
        ],
    )
    def deg(cidx_hbm, p_hbm, cidx_v, buf, acc):
        c = lax.axis_index("c")
        s = lax.axis_index("s")
        wid = c * _NSUB + s
        base = s * _RPS

        def zrow(i, carry):
            for k in range(8):
                buf[i, pl.ds(k * 16, 16)] = jnp.zeros((16,), jnp.float32)
            return carry
        lax.fori_loop(0, 128, zrow, 0)

        def zcp(i, carry):
            pltpu.sync_copy(buf, acc.at[pl.ds(base + i * 128, 128)])
            return carry
        lax.fori_loop(0, _RPS // 128, zcp, 0)
        plsc.subcore_barrier()

        def orow(i, carry):
            for k in range(8):
                buf[i, pl.ds(k * 16, 16)] = jnp.ones((16,), jnp.float32)
            return carry
        lax.fori_loop(0, 128, orow, 0)
        pltpu.sync_copy(cidx_hbm.at[wid], cidx_v)

        def body(b, carry):
            pltpu.sync_copy(buf, acc.at[cidx_v.at[b]], add=True)
            return carry
        lax.fori_loop(0, _NB, body, 0)
        plsc.subcore_barrier()
        pltpu.sync_copy(acc.at[pl.ds(base, _RPS)],
                        p_hbm.at[c, pl.ds(base, _RPS)])

    return deg(cidx)


_BLK = 1024


def _tc_first(x, w, dp):
    def body(x_ref, w_ref, dp0_ref, dp1_ref, z_ref, dis_ref):
        dis = lax.rsqrt(dp0_ref[...] + dp1_ref[...] + 1.0)
        dis_ref[...] = dis
        z_ref[...] = dis * jnp.dot(x_ref[...], w_ref[...],
                                   preferred_element_type=jnp.float32)
    blk = pl.BlockSpec((_BLK, 128), lambda i: (i, 0))
    wblk = pl.BlockSpec((128, 128), lambda i: (0, 0))
    return pl.pallas_call(
        body,
        grid=(_NPAD // _BLK,),
        in_specs=[blk, wblk, blk, blk],
        out_specs=[blk, blk],
        out_shape=[jax.ShapeDtypeStruct((_NPAD, 128), jnp.float32)] * 2,
    )(x, w, dp[0], dp[1])


def _tc_mid(p0, p1, z, dis, b, w, dn):
    def body(p0_ref, p1_ref, z_ref, dis_ref, disn_ref, b_ref, w_ref, out_ref):
        h = dis_ref[...] * (p0_ref[...] + p1_ref[...] + z_ref[...]) + b_ref[...]
        h = jnp.maximum(h, 0.0)
        out_ref[...] = disn_ref[...] * jnp.dot(h, w_ref[...],
                                               preferred_element_type=jnp.float32)
    blk = pl.BlockSpec((_BLK, 128), lambda i: (i, 0))
    blkn = pl.BlockSpec((_BLK, dn), lambda i: (i, 0))
    return pl.pallas_call(
        body,
        grid=(_NPAD // _BLK,),
        in_specs=[blk, blk, blk, blk, blkn,
                  pl.BlockSpec((1, 128), lambda i: (0, 0)),
                  pl.BlockSpec((128, dn), lambda i: (0, 0))],
        out_specs=blkn,
        out_shape=jax.ShapeDtypeStruct((_NPAD, dn), jnp.float32),
    )(p0, p1, z, dis, dis[:, :dn], b, w)


def _tc_final(p0, p1, z, dis, b):
    def body(p0_ref, p1_ref, z_ref, dis_ref, b_ref, out_ref):
        out_ref[...] = dis_ref[...] * (p0_ref[...] + p1_ref[...] + z_ref[...]) \
            + b_ref[...]
    blk = pl.BlockSpec((_BLK, 128), lambda i: (i, 0))
    return pl.pallas_call(
        body,
        grid=(_NPAD // _BLK,),
        in_specs=[blk, blk, blk, blk, pl.BlockSpec((1, 128), lambda i: (0, 0))],
        out_specs=blk,
        out_shape=jax.ShapeDtypeStruct((_NPAD, 128), jnp.float32),
    )(p0, p1, z, dis, b)


def kernel(x, edge_index, W0, b0, W1, b1, W2, b2, W3, b3, W4, b4, W5, b5):
    x = jnp.pad(x, ((0, _NPAD - _N), (0, 0)))
    src = edge_index[0]
    dst = edge_index[1]

    col = dst.reshape(_NW, _EPT)
    pad_c = jnp.full((_NW, _EPT_PAD - _EPT), _TRASH, jnp.int32)
    cidx = jnp.concatenate([col, pad_c], axis=1).reshape(_NW, _NB, 128)

    shalf = src // _NH
    dhalf = dst // _NH
    key = dhalf * 2 + shalf
    rank = jnp.zeros((_E,), jnp.int32)
    counts = []
    for kk in range(4):
        mask = key == kk
        csum = jnp.cumsum(mask.astype(jnp.int32))
        rank = jnp.where(mask, csum - 1, rank)
        counts.append(csum[-1])
    counts = jnp.stack(counts)
    m = ((counts + (_BPB - 1)) // _BPB) * _BPB
    off = jnp.concatenate([jnp.zeros((1,), m.dtype), jnp.cumsum(m)[:3]])
    slot = off[key] + rank
    src_l = src - shalf * _NH
    dst_l = dst - dhalf * _NH
    ridx2 = jnp.zeros((_TOT,), jnp.int32).at[slot].set(src_l)
    cidx2 = jnp.full((_TOT,), _TRASH_L, jnp.int32).at[slot].set(dst_l)
    ridx2 = ridx2.reshape(_TOTB, 128)
    cidx2 = cidx2.reshape(_TOTB, 128)
    meta = (jnp.zeros((128,), jnp.int32)
            .at[0:4].set((m // 1024).astype(jnp.int32))
            .at[4:8].set((off // 1024).astype(jnp.int32)))

    dp = _sc_deg(cidx)
    z, dis = _tc_first(x, W0, dp)

    Ws = [W1, W2, W3, W4]
    bs = [b0, b1, b2, b3]
    for i in range(4):
        p = _sc_agg(z, ridx2, cidx2, meta)
        z = _tc_mid(p[0], p[1], z, dis, bs[i].reshape(1, 128), Ws[i], 128)

    p = _sc_agg(z, ridx2, cidx2, meta)
    W5p = jnp.pad(W5, ((0, 0), (0, 124)))
    z = _tc_mid(p[0], p[1], z, dis, b4.reshape(1, 128), W5p, 128)

    p = _sc_agg(z, ridx2, cidx2, meta)
    b5p = jnp.pad(b5, (0, 124)).reshape(1, 128)
    out = _tc_final(p[0], p[1], z, dis, b5p)
    return out[:_N, :4]


# speedup vs baseline: 1.3895x; 1.3895x over previous
import functools

import jax
import jax.numpy as jnp
from jax import lax
from jax.experimental import pallas as pl
from jax.experimental.pallas import tpu as pltpu
from jax.experimental.pallas import tpu_sc as plsc

_N = 10000
_NPAD = 10240
_E = 320000
_TRASH = _NPAD - 1
_NCORE = 2
_NSUB = 16
_NW = _NCORE * _NSUB
_EPT = _E // _NW
_EPT_PAD = 10240
_NB = _EPT_PAD // 128
_CH = 40
_NCH = _NB // _CH
_RPS = _NPAD // _NSUB

_NH = _NPAD // 2
_ACC_R = _NH + 128
_TRASH_L = _NH
_BPB = 4096
_TOTB = 2636
_TOT = _TOTB * 128
_ZPS = _NH // _NSUB


def _sc_agg(z, ridx2, cidx2, meta):
    mesh = plsc.VectorSubcoreMesh(core_axis_name="c", subcore_axis_name="s")

    @functools.partial(
        pl.kernel,
        out_type=jax.ShapeDtypeStruct((_NCORE, _NPAD, 128), jnp.float32),
        mesh=mesh,
        scratch_types=[
            pltpu.VMEM((128,), jnp.int32),
            pltpu.VMEM((8, 128), jnp.int32),
            pltpu.VMEM((8, 128), jnp.int32),
            pltpu.VMEM((128, 128), jnp.float32),
            pltpu.VMEM_SHARED((_NH, 128), jnp.float32),
            pltpu.VMEM_SHARED((_ACC_R, 128), jnp.float32),
        ],
    )
    def agg(z_hbm, ridx_hbm, cidx_hbm, meta_hbm,
            p_hbm, meta_v, ridx_v, cidx_v, buf, zsh, acc):
        c = lax.axis_index("c")
        s = lax.axis_index("s")
        wid = c * _NSUB + s

        pltpu.sync_copy(meta_hbm, meta_v)

        def zero_buf():
            def zrow(i, carry):
                for k in range(8):
                    buf[i, pl.ds(k * 16, 16)] = jnp.zeros((16,), jnp.float32)
                return carry
            lax.fori_loop(0, 128, zrow, 0)

        def zero_acc():
            for i in range(3):
                blk = i * _NSUB + s

                @pl.when(blk < _ACC_R // 128)
                def _():
                    pltpu.sync_copy(
                        buf, acc.at[pl.ds(pl.multiple_of(blk * 128, 128), 128)])

        zero_buf()
        zero_acc()
        plsc.subcore_barrier()

        zofs = pl.multiple_of(s * _ZPS, 8)
        for k in range(4):
            shalf, dhalf = k % 2, k // 2
            pltpu.sync_copy(
                z_hbm.at[pl.ds(pl.multiple_of(shalf * _NH + s * _ZPS, 8),
                               _ZPS)],
                zsh.at[pl.ds(zofs, _ZPS)])
            plsc.subcore_barrier()

            mv = meta_v[pl.ds(0, 16)]
            nch = mv[k]
            offc = mv[4 + k]
            nloc = (nch - wid + 31) // 32

            def chunk(i, carry):
                base = pl.multiple_of((offc + wid + i * 32) * 8, 8)
                pltpu.sync_copy(ridx_hbm.at[pl.ds(base, 8)], ridx_v)
                pltpu.sync_copy(cidx_hbm.at[pl.ds(base, 8)], cidx_v)
                for j in range(8):
                    pltpu.sync_copy(zsh.at[ridx_v.at[j]], buf)
                    pltpu.sync_copy(buf, acc.at[cidx_v.at[j]], add=True)
                return carry
            lax.fori_loop(0, nloc, chunk, 0)
            plsc.subcore_barrier()

            if k % 2 == 1:
                pltpu.sync_copy(
                    acc.at[pl.ds(zofs, _ZPS)],
                    p_hbm.at[c, pl.ds(pl.multiple_of(
                        dhalf * _NH + s * _ZPS, 8), _ZPS)])
                if k == 1:
                    plsc.subcore_barrier()
                    zero_buf()
                    zero_acc()

    return agg(z, ridx2, cidx2, meta)


def _sc_deg(cidx):
    mesh = plsc.VectorSubcoreMesh(core_axis_name="c", subcore_axis_name="s")

    @functools.partial(
        pl.kernel,
        out_type=jax.ShapeDtypeStruct((_NCORE, _NPAD, 128), jnp.float32),
        mesh=mesh,
        scratch_types=[
            pltpu.VMEM((_NB, 128), jnp.int32),
            pltpu.VMEM((128, 128), jnp.float32),
            pltpu.VMEM_SHARED((_NPAD, 128), jnp.float32),
        ],
    )
    def deg(cidx_hbm, p_hbm, cidx_v, buf, acc):
        c = lax.axis_index("c")
        s = lax.axis_index("s")
        wid = c * _NSUB + s
        base = s * _RPS

        def zrow(i, carry):
            for k in range(8):
                buf[i, pl.ds(k * 16, 16)] = jnp.zeros((16,), jnp.float32)
            return carry
        lax.fori_loop(0, 128, zrow, 0)

        def zcp(i, carry):
            pltpu.sync_copy(buf, acc.at[pl.ds(base + i * 128, 128)])
            return carry
        lax.fori_loop(0, _RPS // 128, zcp, 0)
        plsc.subcore_barrier()

        def orow(i, carry):
            for k in range(8):
                buf[i, pl.ds(k * 16, 16)] = jnp.ones((16,), jnp.float32)
            return carry
        lax.fori_loop(0, 128, orow, 0)
        pltpu.sync_copy(cidx_hbm.at[wid], cidx_v)

        def body(b, carry):
            pltpu.sync_copy(buf, acc.at[cidx_v.at[b]], add=True)
            return carry
        lax.fori_loop(0, _NB, body, 0)
        plsc.subcore_barrier()
        pltpu.sync_copy(acc.at[pl.ds(base, _RPS)],
                        p_hbm.at[c, pl.ds(base, _RPS)])

    return deg(cidx)


_BLK = 1024


def _tc_first(x, w, dp):
    def body(x_ref, w_ref, dp0_ref, dp1_ref, z_ref, dis_ref):
        dis = lax.rsqrt(dp0_ref[...] + dp1_ref[...] + 1.0)
        dis_ref[...] = dis
        z_ref[...] = dis * jnp.dot(x_ref[...], w_ref[...],
                                   preferred_element_type=jnp.float32)
    blk = pl.BlockSpec((_BLK, 128), lambda i: (i, 0))
    wblk = pl.BlockSpec((128, 128), lambda i: (0, 0))
    return pl.pallas_call(
        body,
        grid=(_NPAD // _BLK,),
        in_specs=[blk, wblk, blk, blk],
        out_specs=[blk, blk],
        out_shape=[jax.ShapeDtypeStruct((_NPAD, 128), jnp.float32)] * 2,
    )(x, w, dp[0], dp[1])


def _tc_mid(p0, p1, z, dis, b, w, dn):
    def body(p0_ref, p1_ref, z_ref, dis_ref, disn_ref, b_ref, w_ref, out_ref):
        h = dis_ref[...] * (p0_ref[...] + p1_ref[...] + z_ref[...]) + b_ref[...]
        h = jnp.maximum(h, 0.0)
        out_ref[...] = disn_ref[...] * jnp.dot(h, w_ref[...],
                                               preferred_element_type=jnp.float32)
    blk = pl.BlockSpec((_BLK, 128), lambda i: (i, 0))
    blkn = pl.BlockSpec((_BLK, dn), lambda i: (i, 0))
    return pl.pallas_call(
        body,
        grid=(_NPAD // _BLK,),
        in_specs=[blk, blk, blk, blk, blkn,
                  pl.BlockSpec((1, 128), lambda i: (0, 0)),
                  pl.BlockSpec((128, dn), lambda i: (0, 0))],
        out_specs=blkn,
        out_shape=jax.ShapeDtypeStruct((_NPAD, dn), jnp.float32),
    )(p0, p1, z, dis, dis[:, :dn], b, w)


def _tc_final(p0, p1, z, dis, b):
    def body(p0_ref, p1_ref, z_ref, dis_ref, b_ref, out_ref):
        out_ref[...] = dis_ref[...] * (p0_ref[...] + p1_ref[...] + z_ref[...]) \
            + b_ref[...]
    blk = pl.BlockSpec((_BLK, 128), lambda i: (i, 0))
    return pl.pallas_call(
        body,
        grid=(_NPAD // _BLK,),
        in_specs=[blk, blk, blk, blk, pl.BlockSpec((1, 128), lambda i: (0, 0))],
        out_specs=blk,
        out_shape=jax.ShapeDtypeStruct((_NPAD, 128), jnp.float32),
    )(p0, p1, z, dis, b)


def kernel(x, edge_index, W0, b0, W1, b1, W2, b2, W3, b3, W4, b4, W5, b5):
    x = jnp.pad(x, ((0, _NPAD - _N), (0, 0)))
    src = edge_index[0]
    dst = edge_index[1]

    col = dst.reshape(_NW, _EPT)
    pad_c = jnp.full((_NW, _EPT_PAD - _EPT), _TRASH, jnp.int32)
    cidx = jnp.concatenate([col, pad_c], axis=1).reshape(_NW, _NB, 128)

    shalf = src // _NH
    dhalf = dst // _NH
    key = dhalf * 2 + shalf
    rank = jnp.zeros((_E,), jnp.int32)
    counts = []
    for kk in range(4):
        mask = key == kk
        csum = jnp.cumsum(mask.astype(jnp.int32))
        rank = jnp.where(mask, csum - 1, rank)
        counts.append(csum[-1])
    counts = jnp.stack(counts)
    m = ((counts + (_BPB - 1)) // _BPB) * _BPB
    off = jnp.concatenate([jnp.zeros((1,), m.dtype), jnp.cumsum(m)[:3]])
    slot = off[key] + rank
    src_l = src - shalf * _NH
    dst_l = dst - dhalf * _NH
    packed = (src_l << 13) | dst_l
    flat = jnp.full((_TOT,), _TRASH_L, jnp.int32).at[slot].set(
        packed, unique_indices=True)
    ridx2 = (flat >> 13).reshape(_TOTB, 128)
    cidx2 = (flat & 0x1FFF).reshape(_TOTB, 128)
    meta = (jnp.zeros((128,), jnp.int32)
            .at[0:4].set((m // 1024).astype(jnp.int32))
            .at[4:8].set((off // 1024).astype(jnp.int32)))

    dp = _sc_deg(cidx)
    z, dis = _tc_first(x, W0, dp)

    Ws = [W1, W2, W3, W4]
    bs = [b0, b1, b2, b3]
    for i in range(4):
        p = _sc_agg(z, ridx2, cidx2, meta)
        z = _tc_mid(p[0], p[1], z, dis, bs[i].reshape(1, 128), Ws[i], 128)

    p = _sc_agg(z, ridx2, cidx2, meta)
    W5p = jnp.pad(W5, ((0, 0), (0, 124)))
    z = _tc_mid(p[0], p[1], z, dis, b4.reshape(1, 128), W5p, 128)

    p = _sc_agg(z, ridx2, cidx2, meta)
    b5p = jnp.pad(b5, (0, 124)).reshape(1, 128)
    out = _tc_final(p[0], p[1], z, dis, b5p)
    return out[:_N, :4]


# speedup vs baseline: 1.5792x; 1.1365x over previous
import functools

import jax
import jax.numpy as jnp
from jax import lax
from jax.experimental import pallas as pl
from jax.experimental.pallas import tpu as pltpu
from jax.experimental.pallas import tpu_sc as plsc

_N = 10000
_NPAD = 10240
_E = 320000
_TRASH = _NPAD - 1
_NCORE = 2
_NSUB = 16
_NW = _NCORE * _NSUB
_EPT = _E // _NW
_EPT_PAD = 10240
_NB = _EPT_PAD // 128
_CH = 40
_NCH = _NB // _CH
_RPS = _NPAD // _NSUB


def _sc_agg(z, ridx, cidx, D):
    mesh = plsc.VectorSubcoreMesh(core_axis_name="c", subcore_axis_name="s")

    @functools.partial(
        pl.kernel,
        out_type=jax.ShapeDtypeStruct((_NCORE, _NPAD, D), jnp.float32),
        mesh=mesh,
        scratch_types=[
            pltpu.VMEM((_CH, 128), jnp.int32),
            pltpu.VMEM((_CH, 128), jnp.int32),
            pltpu.VMEM((128, D), jnp.float32),
            pltpu.VMEM((128, D), jnp.float32),
            pltpu.VMEM_SHARED((_NPAD, D), jnp.float32),
            pltpu.SemaphoreType.DMA,
            pltpu.SemaphoreType.DMA,
            pltpu.SemaphoreType.DMA,
            pltpu.SemaphoreType.DMA,
        ],
    )
    def agg(z_hbm, ridx_hbm, cidx_hbm, p_hbm,
            ridx_v, cidx_v, buf0, buf1, acc, sem0, sem1, sem2, sem3):
        c = lax.axis_index("c")
        s = lax.axis_index("s")
        wid = c * _NSUB + s
        base = s * _RPS

        def zrow(i, carry):
            for k in range(D // 16):
                buf0[i, pl.ds(k * 16, 16)] = jnp.zeros((16,), jnp.float32)
            return carry
        lax.fori_loop(0, 128, zrow, 0)

        def zcp(i, carry):
            pltpu.sync_copy(buf0, acc.at[pl.ds(base + i * 128, 128)])
            return carry
        lax.fori_loop(0, _RPS // 128, zcp, 0)
        plsc.subcore_barrier()

        bufs = (buf0, buf1)
        semsa = (sem0, sem1)
        semsb = (sem2, sem3)

        def gat(b, buf, sema, semb):
            pltpu.async_copy(z_hbm.at[ridx_v.at[b, pl.ds(0, 64)]],
                             buf.at[pl.ds(0, 64)], sema)
            pltpu.async_copy(z_hbm.at[ridx_v.at[b, pl.ds(64, 64)]],
                             buf.at[pl.ds(64, 64)], semb)

        def chunk(cc, carry):
            pltpu.sync_copy(ridx_hbm.at[wid, pl.ds(cc * _CH, _CH)], ridx_v)
            pltpu.sync_copy(cidx_hbm.at[wid, pl.ds(cc * _CH, _CH)], cidx_v)
            gat(0, buf0, sem0, sem2)
            gat(1, buf1, sem1, sem3)
            for b in range(_CH):
                buf = bufs[b % 2]
                sema, semb = semsa[b % 2], semsb[b % 2]
                pltpu.make_async_copy(z_hbm.at[ridx_v.at[b, pl.ds(0, 64)]],
                                      buf.at[pl.ds(0, 64)], sema).wait()
                pltpu.make_async_copy(z_hbm.at[ridx_v.at[b, pl.ds(64, 64)]],
                                      buf.at[pl.ds(64, 64)], semb).wait()
                pltpu.sync_copy(buf, acc.at[cidx_v.at[b]], add=True)
                if b + 2 < _CH:
                    gat(b + 2, buf, sema, semb)
            return carry
        lax.fori_loop(0, _NCH, chunk, 0)
        plsc.subcore_barrier()
        pltpu.sync_copy(acc.at[pl.ds(base, _RPS)],
                        p_hbm.at[c, pl.ds(base, _RPS)])

    return agg(z, ridx, cidx)


def _sc_deg(cidx):
    mesh = plsc.VectorSubcoreMesh(core_axis_name="c", subcore_axis_name="s")

    @functools.partial(
        pl.kernel,
        out_type=jax.ShapeDtypeStruct((_NCORE, _NPAD, 128), jnp.float32),
        mesh=mesh,
        scratch_types=[
            pltpu.VMEM((_NB, 128), jnp.int32),
            pltpu.VMEM((128, 128), jnp.float32),
            pltpu.VMEM_SHARED((_NPAD, 128), jnp.float32),
        ],
    )
    def deg(cidx_hbm, p_hbm, cidx_v, buf, acc):
        c = lax.axis_index("c")
        s = lax.axis_index("s")
        wid = c * _NSUB + s
        base = s * _RPS

        def zrow(i, carry):
            for k in range(8):
                buf[i, pl.ds(k * 16, 16)] = jnp.zeros((16,), jnp.float32)
            return carry
        lax.fori_loop(0, 128, zrow, 0)

        def zcp(i, carry):
            pltpu.sync_copy(buf, acc.at[pl.ds(base + i * 128, 128)])
            return carry
        lax.fori_loop(0, _RPS // 128, zcp, 0)
        plsc.subcore_barrier()

        def orow(i, carry):
            for k in range(8):
                buf[i, pl.ds(k * 16, 16)] = jnp.ones((16,), jnp.float32)
            return carry
        lax.fori_loop(0, 128, orow, 0)
        pltpu.sync_copy(cidx_hbm.at[wid], cidx_v)

        def body(b, carry):
            pltpu.sync_copy(buf, acc.at[cidx_v.at[b]], add=True)
            return carry
        lax.fori_loop(0, _NB, body, 0)
        plsc.subcore_barrier()
        pltpu.sync_copy(acc.at[pl.ds(base, _RPS)],
                        p_hbm.at[c, pl.ds(base, _RPS)])

    return deg(cidx)


_BLK = 1024


def _tc_first(x, w, dp):
    def body(x_ref, w_ref, dp0_ref, dp1_ref, z_ref, dis_ref):
        dis = lax.rsqrt(dp0_ref[...] + dp1_ref[...] + 1.0)
        dis_ref[...] = dis
        z_ref[...] = dis * jnp.dot(x_ref[...], w_ref[...],
                                   preferred_element_type=jnp.float32)
    blk = pl.BlockSpec((_BLK, 128), lambda i: (i, 0))
    wblk = pl.BlockSpec((128, 128), lambda i: (0, 0))
    return pl.pallas_call(
        body,
        grid=(_NPAD // _BLK,),
        in_specs=[blk, wblk, blk, blk],
        out_specs=[blk, blk],
        out_shape=[jax.ShapeDtypeStruct((_NPAD, 128), jnp.float32)] * 2,
    )(x, w, dp[0], dp[1])


def _tc_mid(p0, p1, z, dis, b, w, dn):
    def body(p0_ref, p1_ref, z_ref, dis_ref, disn_ref, b_ref, w_ref, out_ref):
        h = dis_ref[...] * (p0_ref[...] + p1_ref[...] + z_ref[...]) + b_ref[...]
        h = jnp.maximum(h, 0.0)
        out_ref[...] = disn_ref[...] * jnp.dot(h, w_ref[...],
                                               preferred_element_type=jnp.float32)
    blk = pl.BlockSpec((_BLK, 128), lambda i: (i, 0))
    blkn = pl.BlockSpec((_BLK, dn), lambda i: (i, 0))
    return pl.pallas_call(
        body,
        grid=(_NPAD // _BLK,),
        in_specs=[blk, blk, blk, blk, blkn,
                  pl.BlockSpec((1, 128), lambda i: (0, 0)),
                  pl.BlockSpec((128, dn), lambda i: (0, 0))],
        out_specs=blkn,
        out_shape=jax.ShapeDtypeStruct((_NPAD, dn), jnp.float32),
    )(p0, p1, z, dis, dis[:, :dn], b, w)


def _tc_final(p0, p1, z, dis, b):
    def body(p0_ref, p1_ref, z_ref, dis_ref, b_ref, out_ref):
        out_ref[...] = dis_ref[...] * (p0_ref[...] + p1_ref[...] + z_ref[...]) \
            + b_ref[...]
    blk = pl.BlockSpec((_BLK, 128), lambda i: (i, 0))
    return pl.pallas_call(
        body,
        grid=(_NPAD // _BLK,),
        in_specs=[blk, blk, blk, blk, pl.BlockSpec((1, 128), lambda i: (0, 0))],
        out_specs=blk,
        out_shape=jax.ShapeDtypeStruct((_NPAD, 128), jnp.float32),
    )(p0, p1, z, dis, b)


def kernel(x, edge_index, W0, b0, W1, b1, W2, b2, W3, b3, W4, b4, W5, b5):
    x = jnp.pad(x, ((0, _NPAD - _N), (0, 0)))
    row = edge_index[0].reshape(_NW, _EPT)
    col = edge_index[1].reshape(_NW, _EPT)
    pad_r = jnp.zeros((_NW, _EPT_PAD - _EPT), jnp.int32)
    pad_c = jnp.full((_NW, _EPT_PAD - _EPT), _TRASH, jnp.int32)
    ridx = jnp.concatenate([row, pad_r], axis=1).reshape(_NW, _NB, 128)
    cidx = jnp.concatenate([col, pad_c], axis=1).reshape(_NW, _NB, 128)

    dp = _sc_deg(cidx)
    z, dis = _tc_first(x, W0, dp)

    Ws = [W1, W2, W3, W4]
    bs = [b0, b1, b2, b3]
    for i in range(4):
        p = _sc_agg(z, ridx, cidx, 128)
        z = _tc_mid(p[0], p[1], z, dis, bs[i].reshape(1, 128), Ws[i], 128)

    p = _sc_agg(z, ridx, cidx, 128)
    W5p = jnp.pad(W5, ((0, 0), (0, 124)))
    z = _tc_mid(p[0], p[1], z, dis, b4.reshape(1, 128), W5p, 128)

    p = _sc_agg(z, ridx, cidx, 128)
    b5p = jnp.pad(b5, (0, 124)).reshape(1, 128)
    out = _tc_final(p[0], p[1], z, dis, b5p)
    return out[:_N, :4]
